# Initial kernel scaffold; baseline (speedup 1.0000x reference)
import jax
import jax.numpy as jnp
from jax.experimental import pallas as pl


def kernel(dep_parents, token_embeds, pos_ids, deprel_ids):
    raise NotImplementedError("write your pallas kernel here")


# speedup vs baseline: 7.3856x; 7.3856x over previous
import functools

import jax
import jax.numpy as jnp
from jax.experimental import pallas as pl
from jax.experimental.pallas import tpu as pltpu

_W_COVERAGE = 2.0
_W_DEGREE = 1.5
_W_POS = 0.8
_W_DEPREL = 0.5
_W_MEDOID = 0.5
_TEMPERATURE = 0.7
_K = 3
_POS_W = (2.0, 1.5, 1.0, 0.8, 0.3, 0.5)
_DEP_W = (2.0, 1.5, 1.5, 1.2, 1.0, 0.5)

_LBLK = 8


def _span_kernel(parents_ref, pos_ref, dep_ref, te_ref,
                 sh_ref, hw_ref, hi_ref, *, L, lblk):
    lb = pl.program_id(1)

    parents = parents_ref[0, :, :]
    tt = jax.lax.broadcasted_iota(jnp.int32, (L, L), 1)
    children = jnp.sum((parents.reshape(L, 1) == tt).astype(jnp.float32),
                       axis=0, keepdims=True)
    pos_ids = pos_ref[0, :, :]
    dep_ids = dep_ref[0, :, :]
    posw = jnp.zeros((1, L), jnp.float32)
    depw = jnp.zeros((1, L), jnp.float32)
    for k in range(6):
        posw = posw + jnp.where(pos_ids == k, _POS_W[k], 0.0)
        depw = depw + jnp.where(dep_ids == k, _DEP_W[k], 0.0)
    base = (_W_COVERAGE + _W_DEGREE * children
            + _W_POS * posw + _W_DEPREL * depw)

    l0 = (lb * lblk).astype(jnp.float32)
    li = l0 + jax.lax.broadcasted_iota(jnp.int32, (lblk, 1, 1), 0).astype(jnp.float32)
    ri = jax.lax.broadcasted_iota(jnp.int32, (lblk, L, 1), 1).astype(jnp.float32)
    ti = jax.lax.broadcasted_iota(jnp.int32, (lblk, L, L), 2).astype(jnp.float32)
    in_span = (ti >= li) & (ti <= ri)
    center = (li + ri) * 0.5
    medoid = _W_MEDOID / (1.0 + jnp.abs(ti - center))
    scores = jnp.where(in_span, base.reshape(1, 1, L) + medoid, 0.0)

    x = scores / _TEMPERATURE
    m = jnp.max(x, axis=-1, keepdims=True)
    e = jnp.exp(x - m)
    s = jnp.sum(e, axis=-1, keepdims=True)
    probs = e / s

    te = te_ref[0]
    valid2 = ri >= li
    p2 = probs.reshape(lblk * L, L)
    sh = jax.lax.dot_general(p2, te, (((1,), (1,)), ((), ())),
                             preferred_element_type=jnp.float32)
    sh = sh.reshape(lblk, L, -1)
    sh_ref[0] = jnp.where(valid2, sh, 0.0)

    ti_i = jax.lax.broadcasted_iota(jnp.int32, (lblk, L, L), 2)
    v = probs
    validr = valid2[:, :, 0]
    for k in range(_K):
        mk = jnp.max(v, axis=-1)
        ak = jnp.min(jnp.where(v == mk[..., None], ti_i, L), axis=-1)
        hw_ref[0, :, k, :] = jnp.where(validr, mk, 0.0)
        hi_ref[0, :, k, :] = jnp.where(validr, ak, -1)
        v = jnp.where(ti_i == ak[..., None], -1.0, v)


def kernel(dep_parents, token_embeds, pos_ids, deprel_ids):
    bsz, dim, L = token_embeds.shape
    lblk = _LBLK

    parents3 = dep_parents.reshape(bsz, 1, L)
    pos3 = pos_ids.reshape(bsz, 1, L)
    dep3 = deprel_ids.reshape(bsz, 1, L)

    grid = (bsz, L // lblk)
    out_shapes = (
        jax.ShapeDtypeStruct((bsz, L, L, dim), jnp.float32),
        jax.ShapeDtypeStruct((bsz, L, _K, L), jnp.float32),
        jax.ShapeDtypeStruct((bsz, L, _K, L), jnp.int32),
    )
    in_specs = [
        pl.BlockSpec((1, 1, L), lambda b, lb: (b, 0, 0)),
        pl.BlockSpec((1, 1, L), lambda b, lb: (b, 0, 0)),
        pl.BlockSpec((1, 1, L), lambda b, lb: (b, 0, 0)),
        pl.BlockSpec((1, dim, L), lambda b, lb: (b, 0, 0)),
    ]
    out_specs = (
        pl.BlockSpec((1, lblk, L, dim), lambda b, lb: (b, lb, 0, 0)),
        pl.BlockSpec((1, lblk, _K, L), lambda b, lb: (b, lb, 0, 0)),
        pl.BlockSpec((1, lblk, _K, L), lambda b, lb: (b, lb, 0, 0)),
    )
    sh, hw, hi = pl.pallas_call(
        functools.partial(_span_kernel, L=L, lblk=lblk),
        grid=grid,
        in_specs=in_specs,
        out_specs=out_specs,
        out_shape=out_shapes,
    )(parents3, pos3, dep3, token_embeds)

    head_weights = jnp.transpose(hw, (0, 1, 3, 2))
    head_indices = jnp.transpose(hi, (0, 1, 3, 2))
    return (sh, head_weights, head_indices)
